# Initial kernel scaffold; baseline (speedup 1.0000x reference)
import jax
import jax.numpy as jnp
from jax.experimental import pallas as pl


def kernel(X, W1c, b1c, W2c, b2c, W1s, b1s, W2s, b2s):
    raise NotImplementedError("write your pallas kernel here")


# speedup vs baseline: 72.8282x; 72.8282x over previous
import jax
import jax.numpy as jnp
from jax.experimental import pallas as pl
from jax.experimental.pallas import tpu as pltpu

_TI = 32


def _cluster_sim_kernel(x_ref, w1c_ref, b1c_ref, w2c_ref, b2c_ref,
                        w1s_ref, b1s_ref, w2s_ref, b2s_ref, o_ref,
                        a_ref, ctr_ref, m_ref):
    X = x_ref[0]
    N, E = X.shape
    W1c = w1c_ref[...]
    H = W1c.shape[1]
    a_ref[...] = (jnp.dot(X, W1c[:E], preferred_element_type=jnp.float32)
                  + b1c_ref[...]).astype(jnp.bfloat16)
    ctr_ref[...] = jnp.dot(X, W1c[E:],
                           preferred_element_type=jnp.float32
                           ).T.astype(jnp.bfloat16)
    w2row = w2c_ref[...].astype(jnp.bfloat16)
    b2 = b2c_ref[0, 0]

    def blk(t, carry):
        i0 = pl.multiple_of(t * _TI, _TI)
        a_t = a_ref[pl.ds(i0, _TI), :].T
        ctr = ctr_ref[...]
        rows = []
        for ii in range(_TI):
            acol = jax.lax.slice(a_t, (0, ii), (H, ii + 1))
            x = jnp.maximum(acol + ctr, jnp.bfloat16(0))
            rows.append(jnp.dot(w2row, x,
                                preferred_element_type=jnp.float32))
        m_ref[pl.ds(i0, _TI), :] = jnp.concatenate(rows, axis=0)
        return carry

    jax.lax.fori_loop(0, N // _TI, blk, 0)

    Hd = jnp.maximum(
        jnp.dot(X, w1s_ref[...], preferred_element_type=jnp.float32)
        + b1s_ref[...], 0.0)
    dvec = jax.nn.sigmoid(
        jnp.sum(Hd * w2s_ref[...], axis=-1, keepdims=True)
        + b2s_ref[0, 0])

    M = m_ref[...]
    MT = M.T
    jj = jax.lax.broadcasted_iota(jnp.int32, (N, N), 1)
    ii = jax.lax.broadcasted_iota(jnp.int32, (N, N), 0)
    vals = jax.nn.sigmoid(jnp.where(jj > ii, M, MT) + b2)
    o_ref[0] = jnp.where(jj == ii, dvec, vals)


def kernel(X, W1c, b1c, W2c, b2c, W1s, b1s, W2s, b2s):
    Bb, N, E = X.shape
    H = W1c.shape[1]
    full = lambda shape: pl.BlockSpec(shape, lambda b: (0,) * len(shape))
    return pl.pallas_call(
        _cluster_sim_kernel,
        grid=(Bb,),
        in_specs=[
            pl.BlockSpec((1, N, E), lambda b: (b, 0, 0)),
            full((2 * E, H)),
            full((1, H)),
            full((1, H)),
            full((1, 1)),
            full((E, H)),
            full((1, H)),
            full((1, H)),
            full((1, 1)),
        ],
        out_specs=pl.BlockSpec((1, N, N), lambda b: (b, 0, 0)),
        out_shape=jax.ShapeDtypeStruct((Bb, N, N), jnp.float32),
        scratch_shapes=[
            pltpu.VMEM((N, H), jnp.bfloat16),
            pltpu.VMEM((H, N), jnp.bfloat16),
            pltpu.VMEM((N, N), jnp.float32),
        ],
        compiler_params=pltpu.CompilerParams(
            dimension_semantics=("parallel",)),
    )(X, W1c, b1c.reshape(1, H), W2c.reshape(1, H), b2c.reshape(1, 1),
      W1s, b1s.reshape(1, H), W2s.reshape(1, H), b2s.reshape(1, 1))


# speedup vs baseline: 83.9370x; 1.1525x over previous
import jax
import jax.numpy as jnp
from jax.experimental import pallas as pl
from jax.experimental.pallas import tpu as pltpu

_TI = 64


def _cluster_sim_kernel(x_ref, w1c_ref, b1c_ref, w2c_ref, b2c_ref,
                        w1s_ref, b1s_ref, w2s_ref, b2s_ref, o_ref,
                        a_ref, ctr_ref, m_ref):
    X = x_ref[0]
    N, E = X.shape
    W1c = w1c_ref[...]
    H = W1c.shape[1]
    a_ref[...] = (jnp.dot(X, W1c[:E], preferred_element_type=jnp.float32)
                  + b1c_ref[...]).astype(jnp.bfloat16)
    ctr_ref[...] = jnp.dot(X, W1c[E:],
                           preferred_element_type=jnp.float32
                           ).T.astype(jnp.bfloat16)
    w2row = w2c_ref[...].astype(jnp.bfloat16)
    b2 = b2c_ref[0, 0]

    def blk(t, carry):
        i0 = pl.multiple_of(t * _TI, _TI)
        a_t = a_ref[pl.ds(i0, _TI), :].T
        ctr = ctr_ref[...]
        rows = []
        for ii in range(_TI):
            acol = jax.lax.slice(a_t, (0, ii), (H, ii + 1))
            x = jnp.maximum(acol + ctr, jnp.bfloat16(0))
            rows.append(jnp.dot(w2row, x,
                                preferred_element_type=jnp.float32))
        m_ref[pl.ds(i0, _TI), :] = jnp.concatenate(rows, axis=0)
        return carry

    jax.lax.fori_loop(0, N // _TI, blk, 0)

    Hd = jnp.maximum(
        jnp.dot(X, w1s_ref[...], preferred_element_type=jnp.float32)
        + b1s_ref[...], 0.0)
    dvec = jax.nn.sigmoid(
        jnp.sum(Hd * w2s_ref[...], axis=-1, keepdims=True)
        + b2s_ref[0, 0])

    M = m_ref[...]
    MT = M.T
    jj = jax.lax.broadcasted_iota(jnp.int32, (N, N), 1)
    ii = jax.lax.broadcasted_iota(jnp.int32, (N, N), 0)
    vals = jax.nn.sigmoid(jnp.where(jj > ii, M, MT) + b2)
    o_ref[0] = jnp.where(jj == ii, dvec, vals)


def kernel(X, W1c, b1c, W2c, b2c, W1s, b1s, W2s, b2s):
    Bb, N, E = X.shape
    H = W1c.shape[1]
    full = lambda shape: pl.BlockSpec(shape, lambda b: (0,) * len(shape))
    return pl.pallas_call(
        _cluster_sim_kernel,
        grid=(Bb,),
        in_specs=[
            pl.BlockSpec((1, N, E), lambda b: (b, 0, 0)),
            full((2 * E, H)),
            full((1, H)),
            full((1, H)),
            full((1, 1)),
            full((E, H)),
            full((1, H)),
            full((1, H)),
            full((1, 1)),
        ],
        out_specs=pl.BlockSpec((1, N, N), lambda b: (b, 0, 0)),
        out_shape=jax.ShapeDtypeStruct((Bb, N, N), jnp.float32),
        scratch_shapes=[
            pltpu.VMEM((N, H), jnp.bfloat16),
            pltpu.VMEM((H, N), jnp.bfloat16),
            pltpu.VMEM((N, N), jnp.float32),
        ],
        compiler_params=pltpu.CompilerParams(
            dimension_semantics=("parallel",)),
    )(X, W1c, b1c.reshape(1, H), W2c.reshape(1, H), b2c.reshape(1, 1),
      W1s, b1s.reshape(1, H), W2s.reshape(1, H), b2s.reshape(1, 1))


# speedup vs baseline: 91.4579x; 1.0896x over previous
import jax
import jax.numpy as jnp
from jax.experimental import pallas as pl
from jax.experimental.pallas import tpu as pltpu

_TI = 128


def _cluster_sim_kernel(x_ref, w1c_ref, b1c_ref, w2c_ref, b2c_ref,
                        w1s_ref, b1s_ref, w2s_ref, b2s_ref, o_ref,
                        a_ref, ctr_ref, m_ref):
    X = x_ref[0]
    N, E = X.shape
    W1c = w1c_ref[...]
    H = W1c.shape[1]
    a_ref[...] = (jnp.dot(X, W1c[:E], preferred_element_type=jnp.float32)
                  + b1c_ref[...]).astype(jnp.bfloat16)
    ctr_ref[...] = jnp.dot(X, W1c[E:],
                           preferred_element_type=jnp.float32
                           ).T.astype(jnp.bfloat16)
    w2row = w2c_ref[...].astype(jnp.bfloat16)
    b2 = b2c_ref[0, 0]

    def blk(t, carry):
        i0 = pl.multiple_of(t * _TI, _TI)
        a_t = a_ref[pl.ds(i0, _TI), :].T
        ctr = ctr_ref[...]
        rows = []
        for ii in range(_TI):
            acol = jax.lax.slice(a_t, (0, ii), (H, ii + 1))
            x = jnp.maximum(acol + ctr, jnp.bfloat16(0))
            rows.append(jnp.dot(w2row, x,
                                preferred_element_type=jnp.float32))
        m_ref[pl.ds(i0, _TI), :] = jnp.concatenate(rows, axis=0)
        return carry

    jax.lax.fori_loop(0, N // _TI, blk, 0)

    Hd = jnp.maximum(
        jnp.dot(X, w1s_ref[...], preferred_element_type=jnp.float32)
        + b1s_ref[...], 0.0)
    dvec = jax.nn.sigmoid(
        jnp.sum(Hd * w2s_ref[...], axis=-1, keepdims=True)
        + b2s_ref[0, 0])

    M = m_ref[...]
    MT = M.T
    jj = jax.lax.broadcasted_iota(jnp.int32, (N, N), 1)
    ii = jax.lax.broadcasted_iota(jnp.int32, (N, N), 0)
    vals = jax.nn.sigmoid(jnp.where(jj > ii, M, MT) + b2)
    o_ref[0] = jnp.where(jj == ii, dvec, vals)


def kernel(X, W1c, b1c, W2c, b2c, W1s, b1s, W2s, b2s):
    Bb, N, E = X.shape
    H = W1c.shape[1]
    full = lambda shape: pl.BlockSpec(shape, lambda b: (0,) * len(shape))
    return pl.pallas_call(
        _cluster_sim_kernel,
        grid=(Bb,),
        in_specs=[
            pl.BlockSpec((1, N, E), lambda b: (b, 0, 0)),
            full((2 * E, H)),
            full((1, H)),
            full((1, H)),
            full((1, 1)),
            full((E, H)),
            full((1, H)),
            full((1, H)),
            full((1, 1)),
        ],
        out_specs=pl.BlockSpec((1, N, N), lambda b: (b, 0, 0)),
        out_shape=jax.ShapeDtypeStruct((Bb, N, N), jnp.float32),
        scratch_shapes=[
            pltpu.VMEM((N, H), jnp.bfloat16),
            pltpu.VMEM((H, N), jnp.bfloat16),
            pltpu.VMEM((N, N), jnp.float32),
        ],
        compiler_params=pltpu.CompilerParams(
            dimension_semantics=("parallel",)),
    )(X, W1c, b1c.reshape(1, H), W2c.reshape(1, H), b2c.reshape(1, 1),
      W1s, b1s.reshape(1, H), W2s.reshape(1, H), b2s.reshape(1, 1))


# speedup vs baseline: 95.8009x; 1.0475x over previous
import jax
import jax.numpy as jnp
from jax.experimental import pallas as pl
from jax.experimental.pallas import tpu as pltpu

_TI = 256


def _cluster_sim_kernel(x_ref, w1c_ref, b1c_ref, w2c_ref, b2c_ref,
                        w1s_ref, b1s_ref, w2s_ref, b2s_ref, o_ref,
                        a_ref, ctr_ref, m_ref):
    X = x_ref[0]
    N, E = X.shape
    W1c = w1c_ref[...]
    H = W1c.shape[1]
    a_ref[...] = (jnp.dot(X, W1c[:E], preferred_element_type=jnp.float32)
                  + b1c_ref[...]).astype(jnp.bfloat16)
    ctr_ref[...] = jnp.dot(X, W1c[E:],
                           preferred_element_type=jnp.float32
                           ).T.astype(jnp.bfloat16)
    w2row = w2c_ref[...].astype(jnp.bfloat16)
    b2 = b2c_ref[0, 0]

    def blk(t, carry):
        i0 = pl.multiple_of(t * _TI, _TI)
        a_t = a_ref[pl.ds(i0, _TI), :].T
        ctr = ctr_ref[...]
        rows = []
        for ii in range(_TI):
            acol = jax.lax.slice(a_t, (0, ii), (H, ii + 1))
            x = jnp.maximum(acol + ctr, jnp.bfloat16(0))
            rows.append(jnp.dot(w2row, x,
                                preferred_element_type=jnp.float32))
        m_ref[pl.ds(i0, _TI), :] = jnp.concatenate(rows, axis=0)
        return carry

    jax.lax.fori_loop(0, N // _TI, blk, 0)

    Hd = jnp.maximum(
        jnp.dot(X, w1s_ref[...], preferred_element_type=jnp.float32)
        + b1s_ref[...], 0.0)
    dvec = jax.nn.sigmoid(
        jnp.sum(Hd * w2s_ref[...], axis=-1, keepdims=True)
        + b2s_ref[0, 0])

    M = m_ref[...]
    MT = M.T
    jj = jax.lax.broadcasted_iota(jnp.int32, (N, N), 1)
    ii = jax.lax.broadcasted_iota(jnp.int32, (N, N), 0)
    vals = jax.nn.sigmoid(jnp.where(jj > ii, M, MT) + b2)
    o_ref[0] = jnp.where(jj == ii, dvec, vals)


def kernel(X, W1c, b1c, W2c, b2c, W1s, b1s, W2s, b2s):
    Bb, N, E = X.shape
    H = W1c.shape[1]
    full = lambda shape: pl.BlockSpec(shape, lambda b: (0,) * len(shape))
    return pl.pallas_call(
        _cluster_sim_kernel,
        grid=(Bb,),
        in_specs=[
            pl.BlockSpec((1, N, E), lambda b: (b, 0, 0)),
            full((2 * E, H)),
            full((1, H)),
            full((1, H)),
            full((1, 1)),
            full((E, H)),
            full((1, H)),
            full((1, H)),
            full((1, 1)),
        ],
        out_specs=pl.BlockSpec((1, N, N), lambda b: (b, 0, 0)),
        out_shape=jax.ShapeDtypeStruct((Bb, N, N), jnp.float32),
        scratch_shapes=[
            pltpu.VMEM((N, H), jnp.bfloat16),
            pltpu.VMEM((H, N), jnp.bfloat16),
            pltpu.VMEM((N, N), jnp.float32),
        ],
        compiler_params=pltpu.CompilerParams(
            dimension_semantics=("parallel",)),
    )(X, W1c, b1c.reshape(1, H), W2c.reshape(1, H), b2c.reshape(1, 1),
      W1s, b1s.reshape(1, H), W2s.reshape(1, H), b2s.reshape(1, 1))


# speedup vs baseline: 100.4909x; 1.0490x over previous
import jax
import jax.numpy as jnp
from jax.experimental import pallas as pl
from jax.experimental.pallas import tpu as pltpu

_TI = 512


def _cluster_sim_kernel(x_ref, w1c_ref, b1c_ref, w2c_ref, b2c_ref,
                        w1s_ref, b1s_ref, w2s_ref, b2s_ref, o_ref,
                        a_ref, ctr_ref, m_ref):
    X = x_ref[0]
    N, E = X.shape
    W1c = w1c_ref[...]
    H = W1c.shape[1]
    a_ref[...] = (jnp.dot(X, W1c[:E], preferred_element_type=jnp.float32)
                  + b1c_ref[...]).astype(jnp.bfloat16)
    ctr_ref[...] = jnp.dot(X, W1c[E:],
                           preferred_element_type=jnp.float32
                           ).T.astype(jnp.bfloat16)
    w2row = w2c_ref[...].astype(jnp.bfloat16)
    b2 = b2c_ref[0, 0]

    def blk(t, carry):
        i0 = pl.multiple_of(t * _TI, _TI)
        a_t = a_ref[pl.ds(i0, _TI), :].T
        ctr = ctr_ref[...]
        rows = []
        for ii in range(_TI):
            acol = jax.lax.slice(a_t, (0, ii), (H, ii + 1))
            x = jnp.maximum(acol + ctr, jnp.bfloat16(0))
            rows.append(jnp.dot(w2row, x,
                                preferred_element_type=jnp.float32))
        m_ref[pl.ds(i0, _TI), :] = jnp.concatenate(rows, axis=0)
        return carry

    jax.lax.fori_loop(0, N // _TI, blk, 0)

    Hd = jnp.maximum(
        jnp.dot(X, w1s_ref[...], preferred_element_type=jnp.float32)
        + b1s_ref[...], 0.0)
    dvec = jax.nn.sigmoid(
        jnp.sum(Hd * w2s_ref[...], axis=-1, keepdims=True)
        + b2s_ref[0, 0])

    M = m_ref[...]
    MT = M.T
    jj = jax.lax.broadcasted_iota(jnp.int32, (N, N), 1)
    ii = jax.lax.broadcasted_iota(jnp.int32, (N, N), 0)
    vals = jax.nn.sigmoid(jnp.where(jj > ii, M, MT) + b2)
    o_ref[0] = jnp.where(jj == ii, dvec, vals)


def kernel(X, W1c, b1c, W2c, b2c, W1s, b1s, W2s, b2s):
    Bb, N, E = X.shape
    H = W1c.shape[1]
    full = lambda shape: pl.BlockSpec(shape, lambda b: (0,) * len(shape))
    return pl.pallas_call(
        _cluster_sim_kernel,
        grid=(Bb,),
        in_specs=[
            pl.BlockSpec((1, N, E), lambda b: (b, 0, 0)),
            full((2 * E, H)),
            full((1, H)),
            full((1, H)),
            full((1, 1)),
            full((E, H)),
            full((1, H)),
            full((1, H)),
            full((1, 1)),
        ],
        out_specs=pl.BlockSpec((1, N, N), lambda b: (b, 0, 0)),
        out_shape=jax.ShapeDtypeStruct((Bb, N, N), jnp.float32),
        scratch_shapes=[
            pltpu.VMEM((N, H), jnp.bfloat16),
            pltpu.VMEM((H, N), jnp.bfloat16),
            pltpu.VMEM((N, N), jnp.float32),
        ],
        compiler_params=pltpu.CompilerParams(
            dimension_semantics=("parallel",)),
    )(X, W1c, b1c.reshape(1, H), W2c.reshape(1, H), b2c.reshape(1, 1),
      W1s, b1s.reshape(1, H), W2s.reshape(1, H), b2s.reshape(1, 1))
